# baseline (speedup 1.0000x reference)
import functools

import jax
import jax.numpy as jnp
from jax import lax
from jax.experimental import pallas as pl
from jax.experimental.pallas import tpu as pltpu
from jax.experimental.pallas import tpu_sc as plsc

EMBED_DIM = 64
NUM_CORES = 2
NUM_SUBCORES = 16
NUM_WORKERS = NUM_CORES * NUM_SUBCORES
CHUNK = 128
NBUF = 4


@functools.partial(jax.jit, static_argnames=("n_chunks",))
def _sc_embedding_lookup(ids, weight, *, n_chunks):
    n_groups = n_chunks // NBUF
    rows_total = NUM_WORKERS * n_chunks * CHUNK
    mesh = plsc.VectorSubcoreMesh(core_axis_name="c", subcore_axis_name="s")

    @functools.partial(
        pl.kernel,
        mesh=mesh,
        out_type=jax.ShapeDtypeStruct((rows_total, EMBED_DIM), jnp.float32),
        scratch_types=[
            pltpu.VMEM((n_chunks, CHUNK), jnp.int32),
            pltpu.VMEM((NBUF, CHUNK, EMBED_DIM), jnp.float32),
        ]
        + [pltpu.SemaphoreType.DMA] * (2 * NBUF),
        compiler_params=pltpu.CompilerParams(use_tc_tiling_on_sc=False),
    )
    def body(ids_hbm, w_hbm, out_hbm, idx_v, rows_v, *sems):
        gsem = sems[:NBUF]
        osem = sems[NBUF:]
        wid = lax.axis_index("s") * NUM_CORES + lax.axis_index("c")
        out_base = wid * (n_chunks * CHUNK)

        pltpu.sync_copy(ids_hbm.at[wid], idx_v)

        def fire_gather(j, slot):
            pltpu.async_copy(w_hbm.at[idx_v.at[j]], rows_v.at[slot], gsem[slot])

        def wait_gather(slot):
            pltpu.make_async_copy(
                w_hbm.at[idx_v.at[0]], rows_v.at[slot], gsem[slot]
            ).wait()

        def fire_store(j, slot):
            pltpu.async_copy(
                rows_v.at[slot],
                out_hbm.at[pl.ds(out_base + j * CHUNK, CHUNK)],
                osem[slot],
            )

        def wait_store(slot):
            pltpu.make_async_copy(
                rows_v.at[slot], out_hbm.at[pl.ds(out_base, CHUNK)], osem[slot]
            ).wait()

        for b in range(NBUF - 1):
            fire_gather(b, b)

        wait_gather(0)
        fire_store(0, 0)
        fire_gather(NBUF - 1, NBUF - 1)

        for b in range(1, NBUF):
            wait_gather(b)
            fire_store(b, b)
            wait_store(b - 1)
            fire_gather(b + NBUF - 1, b - 1)

        def group(g, carry):
            for b in range(NBUF):
                j = g * NBUF + b
                prev = (b - 1) % NBUF
                wait_gather(b)
                fire_store(j, b)
                wait_store(prev)
                fire_gather(j + NBUF - 1, prev)
            return carry

        lax.fori_loop(1, n_groups - 1, group, 0)

        j0 = (n_groups - 1) * NBUF
        wait_gather(0)
        fire_store(j0, 0)
        wait_store(NBUF - 1)
        fire_gather(j0 + NBUF - 1, NBUF - 1)
        for b in range(1, NBUF):
            wait_gather(b)
            fire_store(j0 + b, b)
            wait_store(b - 1)
        wait_store(NBUF - 1)

    return body(ids, weight)


def kernel(token_ids, weight):
    B, L = token_ids.shape
    rows = B * L
    assert rows % (NUM_WORKERS * CHUNK) == 0
    n_chunks = rows // (NUM_WORKERS * CHUNK)
    assert n_chunks % NBUF == 0 and n_chunks // NBUF >= 3
    ids = token_ids.astype(jnp.int32).reshape(NUM_WORKERS, n_chunks, CHUNK)
    out = _sc_embedding_lookup(ids, weight, n_chunks=n_chunks)
    return out.reshape(B, L, EMBED_DIM)
